# baseline (speedup 1.0000x reference)
import functools

import jax
import jax.numpy as jnp
from jax import lax
from jax.experimental import pallas as pl
from jax.experimental.pallas import tpu as pltpu
from jax.experimental.pallas import tpu_sc as plsc

N_NODES = 10000
D_FEAT = 128
N_EDGES = 320000

CHUNK = 128
N_CHUNKS = N_EDGES // CHUNK
NW = 32
CHUNKS_FLOOR = N_CHUNKS // NW
CHUNKS_REM = N_CHUNKS % NW
MAXC = CHUNKS_FLOOR + 1
FLOOR_E = CHUNKS_FLOOR * CHUNK
MAXE = MAXC * CHUNK


NBUF = 3
ROWS_PER_TILE = N_NODES // 16
PACK_CHUNK = 25
NWORD = D_FEAT // 2


def _gae_body(z_hbm, sidx_hbm, didx_hbm, out_hbm, packed_hbm,
              packed_sp, sidx_v, didx_v,
              srows0, srows1, srows2,
              drows0, drows1, drows2, out_v,
              zrow_v, pk_v,
              sem_s0, sem_s1, sem_s2,
              sem_d0, sem_d1, sem_d2):
    cid = lax.axis_index("c")
    sid = lax.axis_index("s")
    wid = sid * 2 + cid
    extra = (wid < CHUNKS_REM).astype(jnp.int32)
    n = CHUNKS_FLOOR + extra
    base_e = (wid * CHUNKS_FLOOR + jnp.minimum(wid, CHUNKS_REM)) * CHUNK

    srows = (srows0, srows1, srows2)
    drows = (drows0, drows1, drows2)
    sems = ((sem_s0, sem_d0), (sem_s1, sem_d1), (sem_s2, sem_d2))

    tile_row0 = sid * ROWS_PER_TILE

    def pack_chunk(k, _):
        r0 = tile_row0 + k * PACK_CHUNK
        pltpu.sync_copy(z_hbm.at[pl.ds(r0, PACK_CHUNK)], zrow_v)

        @plsc.parallel_loop(0, PACK_CHUNK, unroll=4)
        def pack_row(r):
            for w in range(D_FEAT // 32):
                a = zrow_v[r, pl.ds(w * 32, 16)]
                b = zrow_v[r, pl.ds(w * 32 + 16, 16)]
                pk = plsc.pack(a, b, format=plsc.PackFormat.INTERLEAVED)
                pk_v[r, pl.ds(w * 16, 16)] = plsc.bitcast(pk, jnp.int32)

        pltpu.sync_copy(pk_v, packed_hbm.at[cid].at[pl.ds(r0, PACK_CHUNK)])
        pltpu.sync_copy(pk_v, packed_sp.at[pl.ds(r0, PACK_CHUNK)])
        return 0

    lax.fori_loop(0, ROWS_PER_TILE // PACK_CHUNK, pack_chunk, 0)
    plsc.subcore_barrier()

    pltpu.sync_copy(sidx_hbm.at[pl.ds(base_e, FLOOR_E)],
                    sidx_v.at[pl.ds(0, FLOOR_E)])
    pltpu.sync_copy(didx_hbm.at[pl.ds(base_e, FLOOR_E)],
                    didx_v.at[pl.ds(0, FLOOR_E)])

    @pl.when(extra == 1)
    def _():
        pltpu.sync_copy(sidx_hbm.at[pl.ds(base_e + FLOOR_E, CHUNK)],
                        sidx_v.at[pl.ds(FLOOR_E, CHUNK)])
        pltpu.sync_copy(didx_hbm.at[pl.ds(base_e + FLOOR_E, CHUNK)],
                        didx_v.at[pl.ds(FLOOR_E, CHUNK)])

    zp = packed_hbm.at[cid]

    def issue(i, b):
        @pl.when(i % 2 == 0)
        def _():
            pltpu.async_copy(zp.at[sidx_v.at[pl.ds(i * CHUNK, CHUNK)]],
                             srows[b], sems[b][0])
            pltpu.async_copy(zp.at[didx_v.at[pl.ds(i * CHUNK, CHUNK)]],
                             drows[b], sems[b][1])

        @pl.when(i % 2 == 1)
        def _():
            pltpu.async_copy(packed_sp.at[sidx_v.at[pl.ds(i * CHUNK, CHUNK)]],
                             srows[b], sems[b][0])
            pltpu.async_copy(packed_sp.at[didx_v.at[pl.ds(i * CHUNK, CHUNK)]],
                             drows[b], sems[b][1])

    def wait(b):
        pltpu.make_async_copy(zp.at[sidx_v.at[pl.ds(0, CHUNK)]],
                              srows[b], sems[b][0]).wait()
        pltpu.make_async_copy(zp.at[didx_v.at[pl.ds(0, CHUNK)]],
                              drows[b], sems[b][1]).wait()

    lane = lax.iota(jnp.int32, 16)

    def compute(i, b):
        sr, dr = srows[b], drows[b]

        @plsc.parallel_loop(0, CHUNK, unroll=8)
        def edge_body(e):
            acc = None
            for c in range(D_FEAT // 32):
                sv = plsc.bitcast(sr[e, pl.ds(c * 16, 16)], jnp.bfloat16)
                dv = plsc.bitcast(dr[e, pl.ds(c * 16, 16)], jnp.bfloat16)
                s0, s1 = plsc.unpack(sv, format=plsc.PackFormat.INTERLEAVED)
                d0, d1 = plsc.unpack(dv, format=plsc.PackFormat.INTERLEAVED)
                p = s0 * d0 + s1 * d1
                acc = p if acc is None else acc + p
            tot = plsc.cumsum(acc)
            plsc.store_compressed(out_v.at[pl.ds(i * CHUNK + e, 16)], tot,
                                  mask=lane == 15)

    for p in range(NBUF - 1):
        @pl.when(p < n)
        def _():
            issue(p, p)

    n_outer = (n + NBUF - 1) // NBUF

    def outer(oi, _):
        for b in range(NBUF):
            i = oi * NBUF + b

            @pl.when(i + NBUF - 1 < n)
            def _():
                issue(i + NBUF - 1, (b + NBUF - 1) % NBUF)

            @pl.when(i < n)
            def _():
                wait(b)
                compute(i, b)
        return 0

    lax.fori_loop(0, n_outer, outer, 0)

    def sig_body(g, _):
        v = out_v[pl.ds(g * 16, 16)]
        out_v[pl.ds(g * 16, 16)] = 1.0 / (1.0 + jnp.exp(-v))
        return 0

    lax.fori_loop(0, (FLOOR_E + extra * CHUNK) // 16, sig_body, 0)

    pltpu.sync_copy(out_v.at[pl.ds(0, FLOOR_E)],
                    out_hbm.at[pl.ds(base_e, FLOOR_E)])

    @pl.when(extra == 1)
    def _():
        pltpu.sync_copy(out_v.at[pl.ds(FLOOR_E, CHUNK)],
                        out_hbm.at[pl.ds(base_e + FLOOR_E, CHUNK)])


@jax.jit
def _gae_sc(z, src_idx, dst_idx):
    mesh = plsc.VectorSubcoreMesh(core_axis_name="c", subcore_axis_name="s")
    kfn = functools.partial(
        pl.kernel,
        mesh=mesh,
        out_type=(jax.ShapeDtypeStruct((N_EDGES,), jnp.float32),
                  jax.ShapeDtypeStruct((2, N_NODES, NWORD), jnp.int32)),
        compiler_params=pltpu.CompilerParams(needs_layout_passes=False,
                                             use_tc_tiling_on_sc=False),
        scratch_types=[
            pltpu.VMEM_SHARED((N_NODES, NWORD), jnp.int32),
            pltpu.VMEM((MAXE,), jnp.int32),
            pltpu.VMEM((MAXE,), jnp.int32),
        ] + [pltpu.VMEM((CHUNK, NWORD), jnp.int32)] * (2 * NBUF)
          + [pltpu.VMEM((MAXE + 16,), jnp.float32)]
          + [pltpu.VMEM((PACK_CHUNK, D_FEAT), jnp.float32),
             pltpu.VMEM((PACK_CHUNK, NWORD), jnp.int32)]
          + [pltpu.SemaphoreType.DMA] * (2 * NBUF),
    )(_gae_body)
    return kfn(z, src_idx, dst_idx)[0]


def kernel(z, edge_index):
    edge_index = edge_index.astype(jnp.int32)
    src_idx = edge_index[0]
    dst_idx = edge_index[1]
    return _gae_sc(z, src_idx, dst_idx)


# speedup vs baseline: 1.1698x; 1.1698x over previous
import functools

import jax
import jax.numpy as jnp
from jax import lax
from jax.experimental import pallas as pl
from jax.experimental.pallas import tpu as pltpu
from jax.experimental.pallas import tpu_sc as plsc

N_NODES = 10000
D_FEAT = 128
N_EDGES = 320000

CHUNK = 128
N_CHUNKS = N_EDGES // CHUNK
NW = 32
CHUNKS_FLOOR = N_CHUNKS // NW
CHUNKS_REM = N_CHUNKS % NW
MAXC = CHUNKS_FLOOR + 1
FLOOR_E = CHUNKS_FLOOR * CHUNK
MAXE = MAXC * CHUNK


NBUF = 4
ROWS_PER_TILE = N_NODES // 16
PACK_CHUNK = 125
NWORD = D_FEAT // 2


def _gae_body(z_hbm, sidx_hbm, didx_hbm, out_hbm, packed_hbm,
              sidx_v, didx_v,
              srows0, srows1, srows2, srows3,
              drows0, drows1, drows2, drows3, out_v,
              zrow_v, pk_v,
              sem_s0, sem_s1, sem_s2, sem_s3,
              sem_d0, sem_d1, sem_d2, sem_d3):
    cid = lax.axis_index("c")
    sid = lax.axis_index("s")
    wid = sid * 2 + cid
    extra = (wid < CHUNKS_REM).astype(jnp.int32)
    n = CHUNKS_FLOOR + extra
    base_e = (wid * CHUNKS_FLOOR + jnp.minimum(wid, CHUNKS_REM)) * CHUNK

    srows = (srows0, srows1, srows2, srows3)
    drows = (drows0, drows1, drows2, drows3)
    sems = ((sem_s0, sem_d0), (sem_s1, sem_d1),
            (sem_s2, sem_d2), (sem_s3, sem_d3))

    tile_row0 = sid * ROWS_PER_TILE

    def pack_chunk(k, _):
        r0 = tile_row0 + k * PACK_CHUNK
        pltpu.sync_copy(z_hbm.at[pl.ds(r0, PACK_CHUNK)], zrow_v)

        @plsc.parallel_loop(0, PACK_CHUNK, unroll=4)
        def pack_row(r):
            for w in range(D_FEAT // 32):
                a = zrow_v[r, pl.ds(w * 32, 16)]
                b = zrow_v[r, pl.ds(w * 32 + 16, 16)]
                pk = plsc.pack(a, b, format=plsc.PackFormat.INTERLEAVED)
                pk_v[r, pl.ds(w * 16, 16)] = plsc.bitcast(pk, jnp.int32)

        pltpu.sync_copy(pk_v, packed_hbm.at[cid].at[pl.ds(r0, PACK_CHUNK)])
        return 0

    lax.fori_loop(0, ROWS_PER_TILE // PACK_CHUNK, pack_chunk, 0)
    plsc.subcore_barrier()

    pltpu.sync_copy(sidx_hbm.at[pl.ds(base_e, FLOOR_E)],
                    sidx_v.at[pl.ds(0, FLOOR_E)])
    pltpu.sync_copy(didx_hbm.at[pl.ds(base_e, FLOOR_E)],
                    didx_v.at[pl.ds(0, FLOOR_E)])

    @pl.when(extra == 1)
    def _():
        pltpu.sync_copy(sidx_hbm.at[pl.ds(base_e + FLOOR_E, CHUNK)],
                        sidx_v.at[pl.ds(FLOOR_E, CHUNK)])
        pltpu.sync_copy(didx_hbm.at[pl.ds(base_e + FLOOR_E, CHUNK)],
                        didx_v.at[pl.ds(FLOOR_E, CHUNK)])

    zp = packed_hbm.at[cid]

    def issue(i, b):
        pltpu.async_copy(zp.at[sidx_v.at[pl.ds(i * CHUNK, CHUNK)]],
                         srows[b], sems[b][0])
        pltpu.async_copy(zp.at[didx_v.at[pl.ds(i * CHUNK, CHUNK)]],
                         drows[b], sems[b][1])

    def wait(b):
        pltpu.make_async_copy(zp.at[sidx_v.at[pl.ds(0, CHUNK)]],
                              srows[b], sems[b][0]).wait()
        pltpu.make_async_copy(zp.at[didx_v.at[pl.ds(0, CHUNK)]],
                              drows[b], sems[b][1]).wait()

    lane = lax.iota(jnp.int32, 16)

    def compute(i, b):
        sr, dr = srows[b], drows[b]

        @plsc.parallel_loop(0, CHUNK, unroll=8)
        def edge_body(e):
            acc = None
            for c in range(D_FEAT // 32):
                sv = plsc.bitcast(sr[e, pl.ds(c * 16, 16)], jnp.bfloat16)
                dv = plsc.bitcast(dr[e, pl.ds(c * 16, 16)], jnp.bfloat16)
                p0, p1 = plsc.unpack(sv * dv,
                                     format=plsc.PackFormat.INTERLEAVED)
                p = p0 + p1
                acc = p if acc is None else acc + p
            tot = plsc.cumsum(acc)
            plsc.store_compressed(out_v.at[pl.ds(i * CHUNK + e, 16)], tot,
                                  mask=lane == 15)

    for p in range(NBUF - 1):
        @pl.when(p < n)
        def _():
            issue(p, p)

    n_outer = (n + NBUF - 1) // NBUF

    def outer(oi, _):
        for b in range(NBUF):
            i = oi * NBUF + b

            @pl.when(i + NBUF - 1 < n)
            def _():
                issue(i + NBUF - 1, (b + NBUF - 1) % NBUF)

            @pl.when(i < n)
            def _():
                wait(b)
                compute(i, b)
        return 0

    lax.fori_loop(0, n_outer, outer, 0)

    def sig_body(g, _):
        v = out_v[pl.ds(g * 16, 16)]
        out_v[pl.ds(g * 16, 16)] = 1.0 / (1.0 + jnp.exp(-v))
        return 0

    lax.fori_loop(0, (FLOOR_E + extra * CHUNK) // 16, sig_body, 0)

    pltpu.sync_copy(out_v.at[pl.ds(0, FLOOR_E)],
                    out_hbm.at[pl.ds(base_e, FLOOR_E)])

    @pl.when(extra == 1)
    def _():
        pltpu.sync_copy(out_v.at[pl.ds(FLOOR_E, CHUNK)],
                        out_hbm.at[pl.ds(base_e + FLOOR_E, CHUNK)])


@jax.jit
def _gae_sc(z, src_idx, dst_idx):
    mesh = plsc.VectorSubcoreMesh(core_axis_name="c", subcore_axis_name="s")
    kfn = functools.partial(
        pl.kernel,
        mesh=mesh,
        out_type=(jax.ShapeDtypeStruct((N_EDGES,), jnp.float32),
                  jax.ShapeDtypeStruct((2, N_NODES, NWORD), jnp.int32)),
        compiler_params=pltpu.CompilerParams(needs_layout_passes=False,
                                             use_tc_tiling_on_sc=False),
        scratch_types=[
            pltpu.VMEM((MAXE,), jnp.int32),
            pltpu.VMEM((MAXE,), jnp.int32),
        ] + [pltpu.VMEM((CHUNK, NWORD), jnp.int32)] * (2 * NBUF)
          + [pltpu.VMEM((MAXE + 16,), jnp.float32)]
          + [pltpu.VMEM((PACK_CHUNK, D_FEAT), jnp.float32),
             pltpu.VMEM((PACK_CHUNK, NWORD), jnp.int32)]
          + [pltpu.SemaphoreType.DMA] * (2 * NBUF),
    )(_gae_body)
    return kfn(z, src_idx, dst_idx)[0]


def kernel(z, edge_index):
    edge_index = edge_index.astype(jnp.int32)
    src_idx = edge_index[0]
    dst_idx = edge_index[1]
    return _gae_sc(z, src_idx, dst_idx)


# speedup vs baseline: 1.4079x; 1.2035x over previous
import functools

import jax
import jax.numpy as jnp
from jax import lax
from jax.experimental import pallas as pl
from jax.experimental.pallas import tpu as pltpu
from jax.experimental.pallas import tpu_sc as plsc

N_NODES = 10000
D_FEAT = 128
N_EDGES = 320000

CHUNK = 128
N_CHUNKS = N_EDGES // CHUNK
NW = 32
CHUNKS_FLOOR = N_CHUNKS // NW
CHUNKS_REM = N_CHUNKS % NW
MAXC = CHUNKS_FLOOR + 1
FLOOR_E = CHUNKS_FLOOR * CHUNK
MAXE = MAXC * CHUNK


NBUF = 4
ROWS_PER_TILE = N_NODES // 16
PACK_CHUNK = 125
NWORD = D_FEAT // 2


def _gae_body(z_hbm, ei_hbm, out_hbm, packed_hbm,
              sidx_v, didx_v,
              srows0, srows1, srows2, srows3,
              drows0, drows1, drows2, drows3, out_v,
              zrow_v, pk_v,
              sem_s0, sem_s1, sem_s2, sem_s3,
              sem_d0, sem_d1, sem_d2, sem_d3, sem_i0, sem_i1):
    cid = lax.axis_index("c")
    sid = lax.axis_index("s")
    wid = sid * 2 + cid
    extra = (wid < CHUNKS_REM).astype(jnp.int32)
    n = CHUNKS_FLOOR + extra
    base_e = (wid * CHUNKS_FLOOR + jnp.minimum(wid, CHUNKS_REM)) * CHUNK

    srows = (srows0, srows1, srows2, srows3)
    drows = (drows0, drows1, drows2, drows3)
    sems = ((sem_s0, sem_d0), (sem_s1, sem_d1),
            (sem_s2, sem_d2), (sem_s3, sem_d3))

    sidx_hbm = ei_hbm.at[0]
    didx_hbm = ei_hbm.at[1]
    cp_i0 = pltpu.async_copy(sidx_hbm.at[pl.ds(base_e, FLOOR_E)],
                             sidx_v.at[pl.ds(0, FLOOR_E)], sem_i0)
    cp_i1 = pltpu.async_copy(didx_hbm.at[pl.ds(base_e, FLOOR_E)],
                             didx_v.at[pl.ds(0, FLOOR_E)], sem_i1)

    tile_row0 = sid * ROWS_PER_TILE

    def pack_chunk(k, _):
        r0 = tile_row0 + k * PACK_CHUNK
        pltpu.sync_copy(z_hbm.at[pl.ds(r0, PACK_CHUNK)], zrow_v)

        @plsc.parallel_loop(0, PACK_CHUNK, unroll=4)
        def pack_row(r):
            for w in range(D_FEAT // 32):
                a = zrow_v[r, pl.ds(w * 32, 16)]
                b = zrow_v[r, pl.ds(w * 32 + 16, 16)]
                pk = plsc.pack(a, b, format=plsc.PackFormat.INTERLEAVED)
                pk_v[r, pl.ds(w * 16, 16)] = plsc.bitcast(pk, jnp.int32)

        pltpu.sync_copy(pk_v, packed_hbm.at[cid].at[pl.ds(r0, PACK_CHUNK)])
        return 0

    lax.fori_loop(0, ROWS_PER_TILE // PACK_CHUNK, pack_chunk, 0)
    plsc.subcore_barrier()

    cp_i0.wait()
    cp_i1.wait()

    @pl.when(extra == 1)
    def _():
        pltpu.sync_copy(sidx_hbm.at[pl.ds(base_e + FLOOR_E, CHUNK)],
                        sidx_v.at[pl.ds(FLOOR_E, CHUNK)])
        pltpu.sync_copy(didx_hbm.at[pl.ds(base_e + FLOOR_E, CHUNK)],
                        didx_v.at[pl.ds(FLOOR_E, CHUNK)])

    zp = packed_hbm.at[cid]

    def issue(i, b):
        pltpu.async_copy(zp.at[sidx_v.at[pl.ds(i * CHUNK, CHUNK)]],
                         srows[b], sems[b][0])
        pltpu.async_copy(zp.at[didx_v.at[pl.ds(i * CHUNK, CHUNK)]],
                         drows[b], sems[b][1])

    def wait(b):
        pltpu.make_async_copy(zp.at[sidx_v.at[pl.ds(0, CHUNK)]],
                              srows[b], sems[b][0]).wait()
        pltpu.make_async_copy(zp.at[didx_v.at[pl.ds(0, CHUNK)]],
                              drows[b], sems[b][1]).wait()

    lane = lax.iota(jnp.int32, 16)

    def compute(i, b):
        sr, dr = srows[b], drows[b]

        @plsc.parallel_loop(0, CHUNK, unroll=8)
        def edge_body(e):
            acc = None
            for c in range(D_FEAT // 32):
                sv = plsc.bitcast(sr[e, pl.ds(c * 16, 16)], jnp.bfloat16)
                dv = plsc.bitcast(dr[e, pl.ds(c * 16, 16)], jnp.bfloat16)
                p0, p1 = plsc.unpack(sv * dv,
                                     format=plsc.PackFormat.INTERLEAVED)
                p = p0 + p1
                acc = p if acc is None else acc + p
            tot = plsc.cumsum(acc)
            plsc.store_compressed(out_v.at[pl.ds(i * CHUNK + e, 16)], tot,
                                  mask=lane == 15)

    for p in range(NBUF - 1):
        @pl.when(p < n)
        def _():
            issue(p, p)

    n_outer = (n + NBUF - 1) // NBUF

    def outer(oi, _):
        for b in range(NBUF):
            i = oi * NBUF + b

            @pl.when(i + NBUF - 1 < n)
            def _():
                issue(i + NBUF - 1, (b + NBUF - 1) % NBUF)

            @pl.when(i < n)
            def _():
                wait(b)
                compute(i, b)
        return 0

    lax.fori_loop(0, n_outer, outer, 0)

    @plsc.parallel_loop(0, FLOOR_E // 16, unroll=8)
    def sig_body(g):
        v = out_v[pl.ds(g * 16, 16)]
        out_v[pl.ds(g * 16, 16)] = 1.0 / (1.0 + jnp.exp(-v))

    @pl.when(extra == 1)
    def _():
        @plsc.parallel_loop(FLOOR_E // 16, MAXE // 16, unroll=8)
        def sig_tail(g):
            v = out_v[pl.ds(g * 16, 16)]
            out_v[pl.ds(g * 16, 16)] = 1.0 / (1.0 + jnp.exp(-v))

    pltpu.sync_copy(out_v.at[pl.ds(0, FLOOR_E)],
                    out_hbm.at[pl.ds(base_e, FLOOR_E)])

    @pl.when(extra == 1)
    def _():
        pltpu.sync_copy(out_v.at[pl.ds(FLOOR_E, CHUNK)],
                        out_hbm.at[pl.ds(base_e + FLOOR_E, CHUNK)])


@jax.jit
def _gae_sc(z, edge_index):
    mesh = plsc.VectorSubcoreMesh(core_axis_name="c", subcore_axis_name="s")
    kfn = functools.partial(
        pl.kernel,
        mesh=mesh,
        out_type=(jax.ShapeDtypeStruct((N_EDGES,), jnp.float32),
                  jax.ShapeDtypeStruct((2, N_NODES, NWORD), jnp.int32)),
        compiler_params=pltpu.CompilerParams(needs_layout_passes=False,
                                             use_tc_tiling_on_sc=False),
        scratch_types=[
            pltpu.VMEM((MAXE,), jnp.int32),
            pltpu.VMEM((MAXE,), jnp.int32),
        ] + [pltpu.VMEM((CHUNK, NWORD), jnp.int32)] * (2 * NBUF)
          + [pltpu.VMEM((MAXE + 16,), jnp.float32)]
          + [pltpu.VMEM((PACK_CHUNK, D_FEAT), jnp.float32),
             pltpu.VMEM((PACK_CHUNK, NWORD), jnp.int32)]
          + [pltpu.SemaphoreType.DMA] * (2 * NBUF + 2),
    )(_gae_body)
    return kfn(z, edge_index)[0]


def kernel(z, edge_index):
    return _gae_sc(z, edge_index.astype(jnp.int32))
